# baseline (speedup 1.0000x reference)
import jax
import jax.numpy as jnp
import numpy as np
from jax.experimental import pallas as pl
from jax.experimental.pallas import tpu as pltpu

CTX = 32
DM = 1024
CB = 64
HEADS = 8
HD = DM // HEADS
N = 256
CIN = 256
FB = 16

_HI = jax.lax.Precision.HIGHEST


def _bf(a):
    return a.astype(jnp.bfloat16)


def _dot(a, b, precision=None):
    return jnp.dot(a, b, preferred_element_type=jnp.float32, precision=precision)


def _dot_t(a, b, precision=None):
    return jax.lax.dot_general(
        a, b, (((1,), (1,)), ((), ())),
        preferred_element_type=jnp.float32, precision=precision)


def _head_kernel(x_ref, cbW_ref, cbb_ref, cb_ref,
                 e1W_ref, e1b_ref, e2W_ref, e2b_ref,
                 k1W_ref, k1b_ref, k2W_ref, k2b_ref,
                 h2_ref, h2k_ref, idx_ref, lsum_ref):
    x = x_ref[...]
    cb = cb_ref[...]
    z = _dot(x, cbW_ref[...], precision=_HI) + cbb_ref[...]
    zn = jnp.sum(z * z, axis=1, keepdims=True)
    cbn = jnp.sum(cb * cb, axis=1)[None, :]
    cross = _dot_t(z, cb, precision=_HI)
    dist = zn + cbn - 2.0 * cross
    mind = jnp.min(dist, axis=1, keepdims=True)
    lane = jax.lax.broadcasted_iota(jnp.int32, dist.shape, 1)
    idx = jnp.min(jnp.where(dist <= mind, lane, CB), axis=1)
    idx_ref[...] = idx[:, None]
    lsum_ref[...] = jnp.sum(mind, keepdims=True)
    bf = jnp.bfloat16
    h1 = jax.nn.relu(_dot(x.astype(bf), e1W_ref[...]) + e1b_ref[...])
    h2_ref[...] = jax.nn.relu(_dot(h1.astype(bf), e2W_ref[...]) + e2b_ref[...]).astype(bf)
    h1k = jax.nn.relu(_dot(cb.astype(bf), k1W_ref[...]) + k1b_ref[...])
    h2k_ref[...] = jax.nn.relu(_dot(h1k.astype(bf), k2W_ref[...]) + k2b_ref[...]).astype(bf)


def _layernorm(h, w, b):
    m = jnp.mean(h, axis=1, keepdims=True)
    v = jnp.mean((h - m) ** 2, axis=1, keepdims=True)
    return (h - m) / jnp.sqrt(v + 1e-5) * w + b


def _proj_kernel(h2_ref, h2k_ref, e3W_ref, e3b_ref, k3W_ref, k3b_ref,
                 elnw_ref, elnb_ref, klnw_ref, klnb_ref,
                 Wq_ref, bq_ref, Wk_ref, bk_ref, Wv_ref, bv_ref,
                 Qs_ref, Ks_ref, Vs_ref):
    bf = jnp.bfloat16
    h3 = _dot(h2_ref[...], e3W_ref[...]) + e3b_ref[0]
    q = _layernorm(h3, elnw_ref[...], elnb_ref[...]).astype(bf)
    Q = (_dot(q, Wq_ref[...]) + bq_ref[...]).astype(bf)
    hk3 = _dot(h2k_ref[...], k3W_ref[...]) + k3b_ref[0]
    kv = _layernorm(hk3, klnw_ref[...], klnb_ref[...]).astype(bf)
    K = (_dot(kv, Wk_ref[...]) + bk_ref[...]).astype(bf)
    V = (_dot(kv, Wv_ref[...]) + bv_ref[...]).astype(bf)
    for h in range(HEADS):
        sl = slice(h * HD, (h + 1) * HD)
        Qs_ref[h, 0] = Q[:, sl]
        Ks_ref[:, h, 0, 0, :] = K[:, sl]
        Vs_ref[:, h, 0, 0, :] = V[:, sl]


def _attn_kernel(idx_sref, qs_ref, ks_ref, vs_ref, out_ref):
    scale = np.float32(1.0 / float(np.sqrt(HD)))
    neg = np.float32(-1e30)
    S = HEADS * CTX
    rh = jax.lax.broadcasted_iota(jnp.int32, (S, S), 0) // CTX
    ch = jax.lax.broadcasted_iota(jnp.int32, (S, S), 1) // CTX
    same_head = rh == ch
    fb = pl.program_id(0)
    for j in range(FB):
        e = idx_sref[fb * FB + j]
        qst = qs_ref[:, :, j, :].reshape(S, HD)
        kst = ks_ref[e].reshape(S, HD)
        vst = vs_ref[e].reshape(S, HD)
        s = _dot_t(qst, kst) * scale
        s = jnp.where(same_head, s, neg)
        m = jnp.max(s, axis=1, keepdims=True)
        p = jnp.exp(s - m)
        w = p / jnp.sum(p, axis=1, keepdims=True)
        o = _dot(w.astype(jnp.bfloat16), vst)
        for h in range(HEADS):
            out_ref[j, :, h * HD:(h + 1) * HD] = o[h * CTX:(h + 1) * CTX, :]


def _run_head(x, p):
    return pl.pallas_call(
        _head_kernel,
        out_shape=(
            jax.ShapeDtypeStruct((N, DM), jnp.bfloat16),
            jax.ShapeDtypeStruct((CB, DM), jnp.bfloat16),
            jax.ShapeDtypeStruct((N, 1), jnp.int32),
            jax.ShapeDtypeStruct((1, 1), jnp.float32),
        ),
    )(x, p["cb_fc_W"], p["cb_fc_b"][None, :], p["codebook"],
      _bf(p["e_p1_W"]), p["e_p1_b"][None, :], _bf(p["e_p2_W"]), p["e_p2_b"][None, :],
      _bf(p["k_p1_W"]), p["k_p1_b"][None, :], _bf(p["k_p2_W"]), p["k_p2_b"][None, :])


def _run_proj(h2, h2k, p):
    full = lambda shape: pl.BlockSpec(shape, lambda c: (0,) * len(shape))
    in_specs = [
        full((N, DM)),
        full((CB, DM)),
        pl.BlockSpec((DM, DM), lambda c: (0, c)),
        pl.BlockSpec((1, 1, DM), lambda c: (c, 0, 0)),
        pl.BlockSpec((DM, DM), lambda c: (0, c)),
        pl.BlockSpec((1, 1, DM), lambda c: (c, 0, 0)),
        full((1, DM)), full((1, DM)),
        full((1, DM)), full((1, DM)),
        full((DM, DM)), full((1, DM)),
        full((DM, DM)), full((1, DM)),
        full((DM, DM)), full((1, DM)),
    ]
    out_specs = (
        pl.BlockSpec((HEADS, 1, N, HD), lambda c: (0, c, 0, 0)),
        pl.BlockSpec((CB, HEADS, 1, 1, HD), lambda c: (0, 0, c, 0, 0)),
        pl.BlockSpec((CB, HEADS, 1, 1, HD), lambda c: (0, 0, c, 0, 0)),
    )
    return pl.pallas_call(
        _proj_kernel,
        grid=(CTX,),
        in_specs=in_specs,
        out_specs=out_specs,
        out_shape=(
            jax.ShapeDtypeStruct((HEADS, CTX, N, HD), jnp.bfloat16),
            jax.ShapeDtypeStruct((CB, HEADS, CTX, 1, HD), jnp.bfloat16),
            jax.ShapeDtypeStruct((CB, HEADS, CTX, 1, HD), jnp.bfloat16),
        ),
    )(h2, h2k,
      _bf(p["e_p3_W"]), p["e_p3_b"].reshape(CTX, 1, DM),
      _bf(p["k_p3_W"]), p["k_p3_b"].reshape(CTX, 1, DM),
      p["e_ln_w"][None, :], p["e_ln_b"][None, :],
      p["k_ln_w"][None, :], p["k_ln_b"][None, :],
      _bf(p["Wq"]), p["bq"][None, :], _bf(p["Wk"]), p["bk"][None, :],
      _bf(p["Wv"]), p["bv"][None, :])


def _run_attn(idx, Qs, Ks, Vs):
    grid_spec = pltpu.PrefetchScalarGridSpec(
        num_scalar_prefetch=1,
        grid=(N // FB,),
        in_specs=[
            pl.BlockSpec((HEADS, CTX, FB, HD), lambda fb, idxr: (0, 0, fb, 0)),
            pl.BlockSpec((CB, HEADS, CTX, 1, HD), lambda fb, idxr: (0, 0, 0, 0, 0)),
            pl.BlockSpec((CB, HEADS, CTX, 1, HD), lambda fb, idxr: (0, 0, 0, 0, 0)),
        ],
        out_specs=pl.BlockSpec((FB, CTX, DM), lambda fb, idxr: (fb, 0, 0)),
    )
    return pl.pallas_call(
        _attn_kernel,
        grid_spec=grid_spec,
        out_shape=jax.ShapeDtypeStruct((N, CTX, DM), jnp.float32),
    )(idx, Qs, Ks, Vs)


def kernel(emo_prompts, params):
    b, f = emo_prompts.shape[0], emo_prompts.shape[1]
    x = emo_prompts.reshape(N, CIN)
    h2, h2k, idx2, lsum = _run_head(x, params)
    Qs, Ks, Vs = _run_proj(h2, h2k, params)
    out = _run_attn(idx2.reshape(N), Qs, Ks, Vs)
    final = out.reshape(b, f, CTX, DM)
    m = lsum[0, 0] / np.float32(N * DM)
    vq_loss = m + 0.25 * m
    return final, vq_loss


# speedup vs baseline: 1.4649x; 1.4649x over previous
import jax
import jax.numpy as jnp
import numpy as np
from jax.experimental import pallas as pl
from jax.experimental.pallas import tpu as pltpu

CTX = 32
DM = 1024
CB = 64
HEADS = 8
HD = DM // HEADS
N = 256
CIN = 256
FB = 16

_HI = jax.lax.Precision.HIGHEST


def _bf(a):
    return a.astype(jnp.bfloat16)


def _dot(a, b, precision=None):
    return jnp.dot(a, b, preferred_element_type=jnp.float32, precision=precision)


def _dot_t(a, b, precision=None):
    return jax.lax.dot_general(
        a, b, (((1,), (1,)), ((), ())),
        preferred_element_type=jnp.float32, precision=precision)


def _head_kernel(x_ref, cbW_ref, cbb_ref, cb_ref,
                 e1W_ref, e1b_ref, e2W_ref, e2b_ref,
                 k1W_ref, k1b_ref, k2W_ref, k2b_ref,
                 h2_ref, h2k_ref, idx_ref, lsum_ref):
    x = x_ref[...]
    cb = cb_ref[...]
    z = _dot(x, cbW_ref[...], precision=_HI) + cbb_ref[...]
    zn = jnp.sum(z * z, axis=1, keepdims=True)
    cbn = jnp.sum(cb * cb, axis=1)[None, :]
    cross = _dot_t(z, cb, precision=_HI)
    dist = zn + cbn - 2.0 * cross
    mind = jnp.min(dist, axis=1, keepdims=True)
    lane = jax.lax.broadcasted_iota(jnp.int32, dist.shape, 1)
    idx = jnp.min(jnp.where(dist <= mind, lane, CB), axis=1)
    idx_ref[...] = idx[:, None]
    lsum_ref[...] = jnp.sum(mind, keepdims=True)
    bf = jnp.bfloat16
    h1 = jax.nn.relu(_dot(x.astype(bf), e1W_ref[...]) + e1b_ref[...])
    h2_ref[...] = jax.nn.relu(_dot(h1.astype(bf), e2W_ref[...]) + e2b_ref[...]).astype(bf)
    h1k = jax.nn.relu(_dot(cb.astype(bf), k1W_ref[...]) + k1b_ref[...])
    h2k_ref[...] = jax.nn.relu(_dot(h1k.astype(bf), k2W_ref[...]) + k2b_ref[...]).astype(bf)


def _layernorm(h, w, b):
    m = jnp.mean(h, axis=1, keepdims=True)
    v = jnp.mean((h - m) ** 2, axis=1, keepdims=True)
    return (h - m) / jnp.sqrt(v + 1e-5) * w + b


def _proj_kernel(h2_ref, h2k_ref, e3W_ref, e3b_ref, k3W_ref, k3b_ref,
                 elnw_ref, elnb_ref, klnw_ref, klnb_ref,
                 Wq_ref, bq_ref, Wk_ref, bk_ref, Wv_ref, bv_ref,
                 Qs_ref, Ks_ref, Vs_ref):
    bf = jnp.bfloat16
    h3 = _dot(h2_ref[...], e3W_ref[...]) + e3b_ref[0]
    q = _layernorm(h3, elnw_ref[...], elnb_ref[...]).astype(bf)
    Q = (_dot(q, Wq_ref[...]) + bq_ref[...]).astype(bf)
    hk3 = _dot(h2k_ref[...], k3W_ref[...]) + k3b_ref[0]
    kv = _layernorm(hk3, klnw_ref[...], klnb_ref[...]).astype(bf)
    K = (_dot(kv, Wk_ref[...]) + bk_ref[...]).astype(bf)
    V = (_dot(kv, Wv_ref[...]) + bv_ref[...]).astype(bf)
    for h in range(HEADS):
        sl = slice(h * HD, (h + 1) * HD)
        Qs_ref[h, 0] = Q[:, sl]
        Ks_ref[:, h, 0, 0, :] = K[:, sl]
        Vs_ref[:, h, 0, 0, :] = V[:, sl]


def _attn_kernel(idx_sref, qs_ref, ks_ref, vs_ref, out_ref):
    scale = np.float32(1.0 / float(np.sqrt(HD)))
    neg = np.float32(-1e30)
    S = HEADS * CTX
    rh = jax.lax.broadcasted_iota(jnp.int32, (S, S), 0) // CTX
    ch = jax.lax.broadcasted_iota(jnp.int32, (S, S), 1) // CTX
    same_head = rh == ch
    fb = pl.program_id(0)
    for j in range(FB):
        e = idx_sref[fb * FB + j]
        qst = qs_ref[:, :, j, :].reshape(S, HD)
        kst = ks_ref[e].reshape(S, HD)
        vst = vs_ref[e].reshape(S, HD)
        s = _dot_t(qst, kst) * scale
        s = jnp.where(same_head, s, neg)
        m = jnp.max(s, axis=1, keepdims=True)
        p = jnp.exp(s - m)
        w = p / jnp.sum(p, axis=1, keepdims=True)
        o = _dot(w.astype(jnp.bfloat16), vst)
        for h in range(HEADS):
            out_ref[j, :, h * HD:(h + 1) * HD] = o[h * CTX:(h + 1) * CTX, :]


def _run_head(x, p):
    return pl.pallas_call(
        _head_kernel,
        out_shape=(
            jax.ShapeDtypeStruct((N, DM), jnp.bfloat16),
            jax.ShapeDtypeStruct((CB, DM), jnp.bfloat16),
            jax.ShapeDtypeStruct((N, 1), jnp.int32),
            jax.ShapeDtypeStruct((1, 1), jnp.float32),
        ),
    )(x, p["cb_fc_W"], p["cb_fc_b"][None, :], p["codebook"],
      p["e_p1_W"], p["e_p1_b"][None, :], p["e_p2_W"], p["e_p2_b"][None, :],
      p["k_p1_W"], p["k_p1_b"][None, :], p["k_p2_W"], p["k_p2_b"][None, :])


def _run_proj(h2, h2k, p):
    full = lambda shape: pl.BlockSpec(shape, lambda c: (0,) * len(shape))
    in_specs = [
        full((N, DM)),
        full((CB, DM)),
        pl.BlockSpec((DM, DM), lambda c: (0, c)),
        pl.BlockSpec((1, 1, DM), lambda c: (c, 0, 0)),
        pl.BlockSpec((DM, DM), lambda c: (0, c)),
        pl.BlockSpec((1, 1, DM), lambda c: (c, 0, 0)),
        full((1, DM)), full((1, DM)),
        full((1, DM)), full((1, DM)),
        full((DM, DM)), full((1, DM)),
        full((DM, DM)), full((1, DM)),
        full((DM, DM)), full((1, DM)),
    ]
    out_specs = (
        pl.BlockSpec((HEADS, 1, N, HD), lambda c: (0, c, 0, 0)),
        pl.BlockSpec((CB, HEADS, 1, 1, HD), lambda c: (0, 0, c, 0, 0)),
        pl.BlockSpec((CB, HEADS, 1, 1, HD), lambda c: (0, 0, c, 0, 0)),
    )
    return pl.pallas_call(
        _proj_kernel,
        grid=(CTX,),
        in_specs=in_specs,
        out_specs=out_specs,
        out_shape=(
            jax.ShapeDtypeStruct((HEADS, CTX, N, HD), jnp.bfloat16),
            jax.ShapeDtypeStruct((CB, HEADS, CTX, 1, HD), jnp.bfloat16),
            jax.ShapeDtypeStruct((CB, HEADS, CTX, 1, HD), jnp.bfloat16),
        ),
    )(h2, h2k,
      p["e_p3_W"], p["e_p3_b"].reshape(CTX, 1, DM),
      p["k_p3_W"], p["k_p3_b"].reshape(CTX, 1, DM),
      p["e_ln_w"][None, :], p["e_ln_b"][None, :],
      p["k_ln_w"][None, :], p["k_ln_b"][None, :],
      p["Wq"], p["bq"][None, :], p["Wk"], p["bk"][None, :],
      p["Wv"], p["bv"][None, :])


def _run_attn(idx, Qs, Ks, Vs):
    grid_spec = pltpu.PrefetchScalarGridSpec(
        num_scalar_prefetch=1,
        grid=(N // FB,),
        in_specs=[
            pl.BlockSpec((HEADS, CTX, FB, HD), lambda fb, idxr: (0, 0, fb, 0)),
            pl.BlockSpec((CB, HEADS, CTX, 1, HD), lambda fb, idxr: (0, 0, 0, 0, 0)),
            pl.BlockSpec((CB, HEADS, CTX, 1, HD), lambda fb, idxr: (0, 0, 0, 0, 0)),
        ],
        out_specs=pl.BlockSpec((FB, CTX, DM), lambda fb, idxr: (fb, 0, 0)),
    )
    return pl.pallas_call(
        _attn_kernel,
        grid_spec=grid_spec,
        out_shape=jax.ShapeDtypeStruct((N, CTX, DM), jnp.float32),
    )(idx, Qs, Ks, Vs)


def kernel(emo_prompts, params):
    b, f = emo_prompts.shape[0], emo_prompts.shape[1]
    x = emo_prompts.reshape(N, CIN)
    h2, h2k, idx2, lsum = _run_head(x, params)
    Qs, Ks, Vs = _run_proj(h2, h2k, params)
    out = _run_attn(idx2.reshape(N), Qs, Ks, Vs)
    final = out.reshape(b, f, CTX, DM)
    m = lsum[0, 0] / np.float32(N * DM)
    vq_loss = m + 0.25 * m
    return final, vq_loss


# speedup vs baseline: 1.8807x; 1.2839x over previous
import jax
import jax.numpy as jnp
import numpy as np
from jax.experimental import pallas as pl
from jax.experimental.pallas import tpu as pltpu

CTX = 32
DM = 1024
CB = 64
HEADS = 8
HD = DM // HEADS
N = 256
CIN = 256
FB = 16

_HI = jax.lax.Precision.HIGHEST


def _bf(a):
    return a.astype(jnp.bfloat16)


def _dot(a, b, precision=None):
    return jnp.dot(a, b, preferred_element_type=jnp.float32, precision=precision)


def _dot_t(a, b, precision=None):
    return jax.lax.dot_general(
        a, b, (((1,), (1,)), ((), ())),
        preferred_element_type=jnp.float32, precision=precision)


def _head_kernel(x_ref, cbW_ref, cbb_ref, cb_ref,
                 e1W_ref, e1b_ref, e2W_ref, e2b_ref,
                 k1W_ref, k1b_ref, k2W_ref, k2b_ref,
                 h2_ref, h2k_ref, idx_ref, lsum_ref):
    x = x_ref[...]
    cb = cb_ref[...]
    z = _dot(x, cbW_ref[...], precision=_HI) + cbb_ref[...]
    zn = jnp.sum(z * z, axis=1, keepdims=True)
    cbn = jnp.sum(cb * cb, axis=1)[None, :]
    cross = _dot_t(z, cb, precision=_HI)
    dist = zn + cbn - 2.0 * cross
    mind = jnp.min(dist, axis=1, keepdims=True)
    lane = jax.lax.broadcasted_iota(jnp.int32, dist.shape, 1)
    idx = jnp.min(jnp.where(dist <= mind, lane, CB), axis=1)
    idx_ref[...] = idx[:, None]
    lsum_ref[...] = jnp.sum(mind, keepdims=True)
    h1 = jax.nn.relu(_dot(x, e1W_ref[...]) + e1b_ref[...])
    h2_ref[...] = jax.nn.relu(_dot(h1, e2W_ref[...]) + e2b_ref[...])
    h1k = jax.nn.relu(_dot(cb, k1W_ref[...]) + k1b_ref[...])
    h2k_ref[...] = jax.nn.relu(_dot(h1k, k2W_ref[...]) + k2b_ref[...])


def _layernorm(h, w, b):
    m = jnp.mean(h, axis=1, keepdims=True)
    v = jnp.mean((h - m) ** 2, axis=1, keepdims=True)
    return (h - m) / jnp.sqrt(v + 1e-5) * w + b


def _proj_kernel(h2_ref, h2k_ref, e3W_ref, e3b_ref, k3W_ref, k3b_ref,
                 elnw_ref, elnb_ref, klnw_ref, klnb_ref,
                 Wq_ref, bq_ref, Wk_ref, bk_ref, Wv_ref, bv_ref,
                 Qs_ref, Ks_ref, Vs_ref):
    h3 = _dot(h2_ref[...], e3W_ref[...]) + e3b_ref[0]
    q = _layernorm(h3, elnw_ref[...], elnb_ref[...])
    Q = _dot(q, Wq_ref[...]) + bq_ref[...]
    hk3 = _dot(h2k_ref[...], k3W_ref[...]) + k3b_ref[0]
    kv = _layernorm(hk3, klnw_ref[...], klnb_ref[...])
    K = _dot(kv, Wk_ref[...]) + bk_ref[...]
    V = _dot(kv, Wv_ref[...]) + bv_ref[...]
    for h in range(HEADS):
        sl = slice(h * HD, (h + 1) * HD)
        Qs_ref[h, 0] = Q[:, sl]
        Ks_ref[:, h, 0, 0, :] = K[:, sl]
        Vs_ref[:, h, 0, 0, :] = V[:, sl]


def _attn_kernel(idx_sref, qs_ref, ks_ref, vs_ref, out_ref):
    scale = np.float32(1.0 / float(np.sqrt(HD)))
    neg = np.float32(-1e30)
    S = HEADS * CTX
    rh = jax.lax.broadcasted_iota(jnp.int32, (S, S), 0) // CTX
    ch = jax.lax.broadcasted_iota(jnp.int32, (S, S), 1) // CTX
    same_head = rh == ch
    fb = pl.program_id(0)
    for j in range(FB):
        e = idx_sref[fb * FB + j]
        qst = qs_ref[:, :, j, :].reshape(S, HD)
        kst = ks_ref[e].reshape(S, HD)
        vst = vs_ref[e].reshape(S, HD)
        s = _dot_t(qst, kst) * scale
        s = jnp.where(same_head, s, neg)
        m = jnp.max(s, axis=1, keepdims=True)
        p = jnp.exp(s - m)
        w = p / jnp.sum(p, axis=1, keepdims=True)
        o = _dot(w, vst)
        for h in range(HEADS):
            out_ref[j, :, h * HD:(h + 1) * HD] = o[h * CTX:(h + 1) * CTX, :]


def _run_head(x, p):
    return pl.pallas_call(
        _head_kernel,
        out_shape=(
            jax.ShapeDtypeStruct((N, DM), jnp.float32),
            jax.ShapeDtypeStruct((CB, DM), jnp.float32),
            jax.ShapeDtypeStruct((N, 1), jnp.int32),
            jax.ShapeDtypeStruct((1, 1), jnp.float32),
        ),
    )(x, p["cb_fc_W"], p["cb_fc_b"][None, :], p["codebook"],
      p["e_p1_W"], p["e_p1_b"][None, :], p["e_p2_W"], p["e_p2_b"][None, :],
      p["k_p1_W"], p["k_p1_b"][None, :], p["k_p2_W"], p["k_p2_b"][None, :])


def _run_proj(h2, h2k, p):
    full = lambda shape: pl.BlockSpec(shape, lambda c: (0,) * len(shape))
    in_specs = [
        full((N, DM)),
        full((CB, DM)),
        pl.BlockSpec((DM, DM), lambda c: (0, c)),
        pl.BlockSpec((1, 1, DM), lambda c: (c, 0, 0)),
        pl.BlockSpec((DM, DM), lambda c: (0, c)),
        pl.BlockSpec((1, 1, DM), lambda c: (c, 0, 0)),
        full((1, DM)), full((1, DM)),
        full((1, DM)), full((1, DM)),
        full((DM, DM)), full((1, DM)),
        full((DM, DM)), full((1, DM)),
        full((DM, DM)), full((1, DM)),
    ]
    out_specs = (
        pl.BlockSpec((HEADS, 1, N, HD), lambda c: (0, c, 0, 0)),
        pl.BlockSpec((CB, HEADS, 1, 1, HD), lambda c: (0, 0, c, 0, 0)),
        pl.BlockSpec((CB, HEADS, 1, 1, HD), lambda c: (0, 0, c, 0, 0)),
    )
    return pl.pallas_call(
        _proj_kernel,
        grid=(CTX,),
        in_specs=in_specs,
        out_specs=out_specs,
        out_shape=(
            jax.ShapeDtypeStruct((HEADS, CTX, N, HD), jnp.float32),
            jax.ShapeDtypeStruct((CB, HEADS, CTX, 1, HD), jnp.float32),
            jax.ShapeDtypeStruct((CB, HEADS, CTX, 1, HD), jnp.float32),
        ),
    )(h2, h2k,
      p["e_p3_W"], p["e_p3_b"].reshape(CTX, 1, DM),
      p["k_p3_W"], p["k_p3_b"].reshape(CTX, 1, DM),
      p["e_ln_w"][None, :], p["e_ln_b"][None, :],
      p["k_ln_w"][None, :], p["k_ln_b"][None, :],
      p["Wq"], p["bq"][None, :], p["Wk"], p["bk"][None, :],
      p["Wv"], p["bv"][None, :])


def _run_attn(idx, Qs, Ks, Vs):
    grid_spec = pltpu.PrefetchScalarGridSpec(
        num_scalar_prefetch=1,
        grid=(N // FB,),
        in_specs=[
            pl.BlockSpec((HEADS, CTX, FB, HD), lambda fb, idxr: (0, 0, fb, 0)),
            pl.BlockSpec((CB, HEADS, CTX, 1, HD), lambda fb, idxr: (0, 0, 0, 0, 0)),
            pl.BlockSpec((CB, HEADS, CTX, 1, HD), lambda fb, idxr: (0, 0, 0, 0, 0)),
        ],
        out_specs=pl.BlockSpec((FB, CTX, DM), lambda fb, idxr: (fb, 0, 0)),
    )
    return pl.pallas_call(
        _attn_kernel,
        grid_spec=grid_spec,
        out_shape=jax.ShapeDtypeStruct((N, CTX, DM), jnp.float32),
    )(idx, Qs, Ks, Vs)


def kernel(emo_prompts, params):
    b, f = emo_prompts.shape[0], emo_prompts.shape[1]
    x = emo_prompts.reshape(N, CIN)
    h2, h2k, idx2, lsum = _run_head(x, params)
    Qs, Ks, Vs = _run_proj(h2, h2k, params)
    out = _run_attn(idx2.reshape(N), Qs, Ks, Vs)
    final = out.reshape(b, f, CTX, DM)
    m = lsum[0, 0] / np.float32(N * DM)
    vq_loss = m + 0.25 * m
    return final, vq_loss


# speedup vs baseline: 1.9659x; 1.0453x over previous
import jax
import jax.numpy as jnp
import numpy as np
from jax.experimental import pallas as pl
from jax.experimental.pallas import tpu as pltpu

CTX = 32
DM = 1024
CB = 64
HEADS = 8
HD = DM // HEADS
N = 256
CIN = 256
FB = 32
CPS = 2

_HI = jax.lax.Precision.HIGHEST


def _bf(a):
    return a.astype(jnp.bfloat16)


def _dot(a, b, precision=None):
    return jnp.dot(a, b, preferred_element_type=jnp.float32, precision=precision)


def _dot_t(a, b, precision=None):
    return jax.lax.dot_general(
        a, b, (((1,), (1,)), ((), ())),
        preferred_element_type=jnp.float32, precision=precision)


def _head_kernel(x_ref, cbW_ref, cbb_ref, cb_ref,
                 e1W_ref, e1b_ref, e2W_ref, e2b_ref,
                 k1W_ref, k1b_ref, k2W_ref, k2b_ref,
                 h2_ref, h2k_ref, idx_ref, lsum_ref):
    x = x_ref[...]
    cb = cb_ref[...]
    z = _dot(x, cbW_ref[...], precision=_HI) + cbb_ref[...]
    zn = jnp.sum(z * z, axis=1, keepdims=True)
    cbn = jnp.sum(cb * cb, axis=1)[None, :]
    cross = _dot_t(z, cb, precision=_HI)
    dist = zn + cbn - 2.0 * cross
    mind = jnp.min(dist, axis=1, keepdims=True)
    lane = jax.lax.broadcasted_iota(jnp.int32, dist.shape, 1)
    idx = jnp.min(jnp.where(dist <= mind, lane, CB), axis=1)
    idx_ref[...] = idx[:, None]
    lsum_ref[...] = jnp.sum(mind, keepdims=True)
    h1 = jax.nn.relu(_dot(x, e1W_ref[...]) + e1b_ref[...])
    h2_ref[...] = jax.nn.relu(_dot(h1, e2W_ref[...]) + e2b_ref[...])
    h1k = jax.nn.relu(_dot(cb, k1W_ref[...]) + k1b_ref[...])
    h2k_ref[...] = jax.nn.relu(_dot(h1k, k2W_ref[...]) + k2b_ref[...])


def _layernorm(h, w, b):
    m = jnp.mean(h, axis=1, keepdims=True)
    v = jnp.mean((h - m) ** 2, axis=1, keepdims=True)
    return (h - m) / jnp.sqrt(v + 1e-5) * w + b


def _proj_kernel(h2_ref, h2k_ref, e3W_ref, e3b_ref, k3W_ref, k3b_ref,
                 elnw_ref, elnb_ref, klnw_ref, klnb_ref,
                 Wq_ref, bq_ref, Wk_ref, bk_ref, Wv_ref, bv_ref,
                 Qs_ref, Ks_ref, Vs_ref):
    h3w = _dot(h2_ref[...], e3W_ref[...])
    hk3w = _dot(h2k_ref[...], k3W_ref[...])
    for cc in range(CPS):
        dsl = slice(cc * DM, (cc + 1) * DM)
        h3 = h3w[:, dsl] + e3b_ref[cc]
        q = _layernorm(h3, elnw_ref[...], elnb_ref[...])
        Q = _dot(q, Wq_ref[...]) + bq_ref[...]
        hk3 = hk3w[:, dsl] + k3b_ref[cc]
        kv = _layernorm(hk3, klnw_ref[...], klnb_ref[...])
        K = _dot(kv, Wk_ref[...]) + bk_ref[...]
        V = _dot(kv, Wv_ref[...]) + bv_ref[...]
        for h in range(HEADS):
            sl = slice(h * HD, (h + 1) * HD)
            Qs_ref[h, cc] = Q[:, sl]
            Ks_ref[:, h, cc, 0, :] = K[:, sl]
            Vs_ref[:, h, cc, 0, :] = V[:, sl]


def _attn_kernel(idx_sref, qs_ref, ks_ref, vs_ref, out_ref):
    scale = np.float32(1.0 / float(np.sqrt(HD)))
    neg = np.float32(-1e30)
    S = HEADS * CTX
    rh = jax.lax.broadcasted_iota(jnp.int32, (S, S), 0) // CTX
    ch = jax.lax.broadcasted_iota(jnp.int32, (S, S), 1) // CTX
    same_head = rh == ch
    fb = pl.program_id(0)
    for j in range(FB):
        e = idx_sref[fb * FB + j]
        qst = qs_ref[:, :, j, :].reshape(S, HD)
        kst = ks_ref[e].reshape(S, HD)
        vst = vs_ref[e].reshape(S, HD)
        s = _dot_t(qst, kst) * scale
        s = jnp.where(same_head, s, neg)
        m = jnp.max(s, axis=1, keepdims=True)
        p = jnp.exp(s - m)
        w = p / jnp.sum(p, axis=1, keepdims=True)
        o = _dot(w, vst)
        for h in range(HEADS):
            out_ref[j, :, h * HD:(h + 1) * HD] = o[h * CTX:(h + 1) * CTX, :]


def _run_head(x, p):
    return pl.pallas_call(
        _head_kernel,
        out_shape=(
            jax.ShapeDtypeStruct((N, DM), jnp.float32),
            jax.ShapeDtypeStruct((CB, DM), jnp.float32),
            jax.ShapeDtypeStruct((N, 1), jnp.int32),
            jax.ShapeDtypeStruct((1, 1), jnp.float32),
        ),
    )(x, p["cb_fc_W"], p["cb_fc_b"][None, :], p["codebook"],
      p["e_p1_W"], p["e_p1_b"][None, :], p["e_p2_W"], p["e_p2_b"][None, :],
      p["k_p1_W"], p["k_p1_b"][None, :], p["k_p2_W"], p["k_p2_b"][None, :])


def _run_proj(h2, h2k, p):
    full = lambda shape: pl.BlockSpec(shape, lambda c: (0,) * len(shape))
    in_specs = [
        full((N, DM)),
        full((CB, DM)),
        pl.BlockSpec((DM, CPS * DM), lambda c: (0, c)),
        pl.BlockSpec((CPS, 1, DM), lambda c: (c, 0, 0)),
        pl.BlockSpec((DM, CPS * DM), lambda c: (0, c)),
        pl.BlockSpec((CPS, 1, DM), lambda c: (c, 0, 0)),
        full((1, DM)), full((1, DM)),
        full((1, DM)), full((1, DM)),
        full((DM, DM)), full((1, DM)),
        full((DM, DM)), full((1, DM)),
        full((DM, DM)), full((1, DM)),
    ]
    out_specs = (
        pl.BlockSpec((HEADS, CPS, N, HD), lambda c: (0, c, 0, 0)),
        pl.BlockSpec((CB, HEADS, CPS, 1, HD), lambda c: (0, 0, c, 0, 0)),
        pl.BlockSpec((CB, HEADS, CPS, 1, HD), lambda c: (0, 0, c, 0, 0)),
    )
    return pl.pallas_call(
        _proj_kernel,
        grid=(CTX // CPS,),
        in_specs=in_specs,
        out_specs=out_specs,
        out_shape=(
            jax.ShapeDtypeStruct((HEADS, CTX, N, HD), jnp.float32),
            jax.ShapeDtypeStruct((CB, HEADS, CTX, 1, HD), jnp.float32),
            jax.ShapeDtypeStruct((CB, HEADS, CTX, 1, HD), jnp.float32),
        ),
    )(h2, h2k,
      p["e_p3_W"], p["e_p3_b"].reshape(CTX, 1, DM),
      p["k_p3_W"], p["k_p3_b"].reshape(CTX, 1, DM),
      p["e_ln_w"][None, :], p["e_ln_b"][None, :],
      p["k_ln_w"][None, :], p["k_ln_b"][None, :],
      p["Wq"], p["bq"][None, :], p["Wk"], p["bk"][None, :],
      p["Wv"], p["bv"][None, :])


def _run_attn(idx, Qs, Ks, Vs):
    grid_spec = pltpu.PrefetchScalarGridSpec(
        num_scalar_prefetch=1,
        grid=(N // FB,),
        in_specs=[
            pl.BlockSpec((HEADS, CTX, FB, HD), lambda fb, idxr: (0, 0, fb, 0)),
            pl.BlockSpec((CB, HEADS, CTX, 1, HD), lambda fb, idxr: (0, 0, 0, 0, 0)),
            pl.BlockSpec((CB, HEADS, CTX, 1, HD), lambda fb, idxr: (0, 0, 0, 0, 0)),
        ],
        out_specs=pl.BlockSpec((FB, CTX, DM), lambda fb, idxr: (fb, 0, 0)),
    )
    return pl.pallas_call(
        _attn_kernel,
        grid_spec=grid_spec,
        out_shape=jax.ShapeDtypeStruct((N, CTX, DM), jnp.float32),
    )(idx, Qs, Ks, Vs)


def kernel(emo_prompts, params):
    b, f = emo_prompts.shape[0], emo_prompts.shape[1]
    x = emo_prompts.reshape(N, CIN)
    h2, h2k, idx2, lsum = _run_head(x, params)
    Qs, Ks, Vs = _run_proj(h2, h2k, params)
    out = _run_attn(idx2.reshape(N), Qs, Ks, Vs)
    final = out.reshape(b, f, CTX, DM)
    m = lsum[0, 0] / np.float32(N * DM)
    vq_loss = m + 0.25 * m
    return final, vq_loss


# speedup vs baseline: 2.2940x; 1.1669x over previous
import jax
import jax.numpy as jnp
import numpy as np
from jax.experimental import pallas as pl
from jax.experimental.pallas import tpu as pltpu

CTX = 32
DM = 1024
CB = 64
HEADS = 8
HD = DM // HEADS
N = 256
CIN = 256
FB = 16
CPS = 2

_HI = jax.lax.Precision.HIGHEST


def _bf(a):
    return a.astype(jnp.bfloat16)


def _dot(a, b, precision=None):
    return jnp.dot(a, b, preferred_element_type=jnp.float32, precision=precision)


def _dot_t(a, b, precision=None):
    return jax.lax.dot_general(
        a, b, (((1,), (1,)), ((), ())),
        preferred_element_type=jnp.float32, precision=precision)


def _head_kernel(x_ref, cbW_ref, cbb_ref, cb_ref,
                 e1W_ref, e1b_ref, e2W_ref, e2b_ref,
                 k1W_ref, k1b_ref, k2W_ref, k2b_ref,
                 h2_ref, h2k_ref, idx_ref, lsum_ref):
    x = x_ref[...]
    cb = cb_ref[...]
    z = _dot(x, cbW_ref[...], precision=_HI) + cbb_ref[...]
    zn = jnp.sum(z * z, axis=1, keepdims=True)
    cbn = jnp.sum(cb * cb, axis=1)[None, :]
    cross = _dot_t(z, cb, precision=_HI)
    dist = zn + cbn - 2.0 * cross
    mind = jnp.min(dist, axis=1, keepdims=True)
    lane = jax.lax.broadcasted_iota(jnp.int32, dist.shape, 1)
    idx = jnp.min(jnp.where(dist <= mind, lane, CB), axis=1)
    idx_ref[...] = idx[:, None]
    lsum_ref[...] = jnp.sum(mind, keepdims=True)
    h1 = jax.nn.relu(_dot(x, e1W_ref[...]) + e1b_ref[...])
    h2_ref[...] = jax.nn.relu(_dot(h1, e2W_ref[...]) + e2b_ref[...])
    h1k = jax.nn.relu(_dot(cb, k1W_ref[...]) + k1b_ref[...])
    h2k_ref[...] = jax.nn.relu(_dot(h1k, k2W_ref[...]) + k2b_ref[...])


def _layernorm(h, w, b):
    m = jnp.mean(h, axis=1, keepdims=True)
    v = jnp.mean((h - m) ** 2, axis=1, keepdims=True)
    return (h - m) / jnp.sqrt(v + 1e-5) * w + b


def _proj_kernel(h2_ref, h2k_ref, e3W_ref, e3b_ref, k3W_ref, k3b_ref,
                 elnw_ref, elnb_ref, klnw_ref, klnb_ref,
                 Wq_ref, bq_ref, Wk_ref, bk_ref, Wv_ref, bv_ref,
                 Qs_ref, Ks_ref, Vs_ref):
    h3w = _dot(h2_ref[...], e3W_ref[...])
    hk3w = _dot(h2k_ref[...], k3W_ref[...])
    for cc in range(CPS):
        dsl = slice(cc * DM, (cc + 1) * DM)
        h3 = h3w[:, dsl] + e3b_ref[cc]
        q = _layernorm(h3, elnw_ref[...], elnb_ref[...])
        Q = _dot(q, Wq_ref[...]) + bq_ref[...]
        hk3 = hk3w[:, dsl] + k3b_ref[cc]
        kv = _layernorm(hk3, klnw_ref[...], klnb_ref[...])
        K = _dot(kv, Wk_ref[...]) + bk_ref[...]
        V = _dot(kv, Wv_ref[...]) + bv_ref[...]
        for h in range(HEADS):
            sl = slice(h * HD, (h + 1) * HD)
            Qs_ref[:, h, cc, 0, :] = Q[:, sl]
            Ks_ref[:, h, cc, 0, :] = K[:, sl]
            Vs_ref[:, h, cc, 0, :] = V[:, sl]


def _attn_kernel(idx_sref, qs_ref, ks_ref, vs_ref, out_ref):
    scale = np.float32(1.0 / float(np.sqrt(HD)))
    neg = np.float32(-1e30)
    S = HEADS * CTX
    rh = jax.lax.broadcasted_iota(jnp.int32, (S, S), 0) // CTX
    ch = jax.lax.broadcasted_iota(jnp.int32, (S, S), 1) // CTX
    same_head = rh == ch
    fb = pl.program_id(0)
    for j in range(FB):
        e = idx_sref[fb * FB + j]
        qst = qs_ref[j]
        kst = ks_ref[e]
        vst = vs_ref[e]
        s = _dot_t(qst, kst) * scale
        s = jnp.where(same_head, s, neg)
        m = jnp.max(s, axis=1, keepdims=True)
        p = jnp.exp(s - m)
        w = p / jnp.sum(p, axis=1, keepdims=True)
        o = _dot(w, vst)
        for h in range(HEADS):
            out_ref[j, :, h * HD:(h + 1) * HD] = o[h * CTX:(h + 1) * CTX, :]


def _run_head(x, p):
    return pl.pallas_call(
        _head_kernel,
        out_shape=(
            jax.ShapeDtypeStruct((N, DM), jnp.float32),
            jax.ShapeDtypeStruct((CB, DM), jnp.float32),
            jax.ShapeDtypeStruct((N, 1), jnp.int32),
            jax.ShapeDtypeStruct((1, 1), jnp.float32),
        ),
    )(x, p["cb_fc_W"], p["cb_fc_b"][None, :], p["codebook"],
      p["e_p1_W"], p["e_p1_b"][None, :], p["e_p2_W"], p["e_p2_b"][None, :],
      p["k_p1_W"], p["k_p1_b"][None, :], p["k_p2_W"], p["k_p2_b"][None, :])


def _run_proj(h2, h2k, p):
    full = lambda shape: pl.BlockSpec(shape, lambda c: (0,) * len(shape))
    in_specs = [
        full((N, DM)),
        full((CB, DM)),
        pl.BlockSpec((DM, CPS * DM), lambda c: (0, c)),
        pl.BlockSpec((CPS, 1, DM), lambda c: (c, 0, 0)),
        pl.BlockSpec((DM, CPS * DM), lambda c: (0, c)),
        pl.BlockSpec((CPS, 1, DM), lambda c: (c, 0, 0)),
        full((1, DM)), full((1, DM)),
        full((1, DM)), full((1, DM)),
        full((DM, DM)), full((1, DM)),
        full((DM, DM)), full((1, DM)),
        full((DM, DM)), full((1, DM)),
    ]
    out_specs = (
        pl.BlockSpec((N, HEADS, CPS, 1, HD), lambda c: (0, 0, c, 0, 0)),
        pl.BlockSpec((CB, HEADS, CPS, 1, HD), lambda c: (0, 0, c, 0, 0)),
        pl.BlockSpec((CB, HEADS, CPS, 1, HD), lambda c: (0, 0, c, 0, 0)),
    )
    return pl.pallas_call(
        _proj_kernel,
        grid=(CTX // CPS,),
        in_specs=in_specs,
        out_specs=out_specs,
        out_shape=(
            jax.ShapeDtypeStruct((N, HEADS, CTX, 1, HD), jnp.float32),
            jax.ShapeDtypeStruct((CB, HEADS, CTX, 1, HD), jnp.float32),
            jax.ShapeDtypeStruct((CB, HEADS, CTX, 1, HD), jnp.float32),
        ),
    )(h2, h2k,
      p["e_p3_W"], p["e_p3_b"].reshape(CTX, 1, DM),
      p["k_p3_W"], p["k_p3_b"].reshape(CTX, 1, DM),
      p["e_ln_w"][None, :], p["e_ln_b"][None, :],
      p["k_ln_w"][None, :], p["k_ln_b"][None, :],
      p["Wq"], p["bq"][None, :], p["Wk"], p["bk"][None, :],
      p["Wv"], p["bv"][None, :])


def _run_attn(idx, Qs, Ks, Vs):
    grid_spec = pltpu.PrefetchScalarGridSpec(
        num_scalar_prefetch=1,
        grid=(N // FB,),
        in_specs=[
            pl.BlockSpec((FB, HEADS * CTX, HD), lambda fb, idxr: (fb, 0, 0)),
            pl.BlockSpec((CB, HEADS * CTX, HD), lambda fb, idxr: (0, 0, 0)),
            pl.BlockSpec((CB, HEADS * CTX, HD), lambda fb, idxr: (0, 0, 0)),
        ],
        out_specs=pl.BlockSpec((FB, CTX, DM), lambda fb, idxr: (fb, 0, 0)),
    )
    return pl.pallas_call(
        _attn_kernel,
        grid_spec=grid_spec,
        out_shape=jax.ShapeDtypeStruct((N, CTX, DM), jnp.float32),
    )(idx, Qs.reshape(N, HEADS * CTX, HD), Ks.reshape(CB, HEADS * CTX, HD),
      Vs.reshape(CB, HEADS * CTX, HD))


def kernel(emo_prompts, params):
    b, f = emo_prompts.shape[0], emo_prompts.shape[1]
    x = emo_prompts.reshape(N, CIN)
    h2, h2k, idx2, lsum = _run_head(x, params)
    Qs, Ks, Vs = _run_proj(h2, h2k, params)
    out = _run_attn(idx2.reshape(N), Qs, Ks, Vs)
    final = out.reshape(b, f, CTX, DM)
    m = lsum[0, 0] / np.float32(N * DM)
    vq_loss = m + 0.25 * m
    return final, vq_loss
